# baseline (speedup 1.0000x reference)
import functools
import jax
import jax.numpy as jnp
from jax import lax
from jax.experimental import pallas as pl
from jax.experimental.pallas import tpu as pltpu
from jax.experimental.pallas import tpu_sc as plsc

N = 16384
D_IN = 64
K = 32
DIM = 128
HEADS = 4
DH = DIM // HEADS
M = 2457
N_BASE = 2048
N_EXTRA = 409
MP = 2560
EP = MP * K
DT = 128
NEG = -1e30
INF = 3e38
BIGI = 2**30


def _fps_body(px_ref, py_ref, pz_ref, idx_ref):
    px = px_ref[...]
    py = py_ref[...]
    pz = pz_ref[...]
    row = lax.broadcasted_iota(jnp.int32, (128, 128), 0)
    col = lax.broadcasted_iota(jnp.int32, (128, 128), 1)
    iota = row * 128 + col

    def run_chain(start, n_sel, out_off):
        sx = px[start // 128, start % 128]
        sy = py[start // 128, start % 128]
        sz = pz[start // 128, start % 128]
        d0 = ((px - sx) ** 2 + (pz - sz) ** 2) + (py - sy) ** 2
        idx_ref[out_off] = jnp.int32(start)

        def it(i, d):
            m = jnp.max(d)
            nxt = jnp.min(jnp.where(d == m, iota, BIGI))
            sel = iota == nxt
            nx = jnp.sum(jnp.where(sel, px, 0.0))
            ny = jnp.sum(jnp.where(sel, py, 0.0))
            nz = jnp.sum(jnp.where(sel, pz, 0.0))
            idx_ref[out_off + i] = nxt
            dn = ((px - nx) ** 2 + (pz - nz) ** 2) + (py - ny) ** 2
            return jnp.minimum(d, dn)

        lax.fori_loop(1, n_sel, it, d0)

    run_chain(0, N_BASE, 0)
    run_chain(N // 2, N_EXTRA, N_BASE)


def _fps(pos):
    px = pos[:, 0].reshape(128, 128)
    py = pos[:, 1].reshape(128, 128)
    pz = pos[:, 2].reshape(128, 128)
    idx = pl.pallas_call(
        _fps_body,
        out_shape=jax.ShapeDtypeStruct((2464,), jnp.int32),
        out_specs=pl.BlockSpec(memory_space=pltpu.SMEM),
    )(px, py, pz)
    return idx[:M]


def _knn_body(pos3_ref, qs_ref, col_ref):
    px = pos3_ref[0, :]
    py = pos3_ref[1, :]
    pz = pos3_ref[2, :]
    qx = qs_ref[:, 0]
    qy = qs_ref[:, 1]
    qz = qs_ref[:, 2]
    qn = (qx * qx + qz * qz) + qy * qy
    pn = (px * px + pz * pz) + py * py
    bf = lambda a: a.astype(jnp.bfloat16).astype(jnp.float32)
    qbx, qby, qbz = bf(qx), bf(qy), bf(qz)
    pbx, pby, pbz = bf(px), bf(py), bf(pz)
    t = (qbx[:, None] * pbx[None, :] + qby[:, None] * pby[None, :]) \
        + qbz[:, None] * pbz[None, :]
    d = (qn[:, None] + pn[None, :]) - 2.0 * t
    d3 = d.reshape(128, K, 512)
    ci3 = lax.broadcasted_iota(jnp.int32, (128, K, 512), 1) * 512 \
        + lax.broadcasted_iota(jnp.int32, (128, K, 512), 2)
    T = 8
    gs, ga = [], []
    cur = d3
    for l in range(T):
        gm = jnp.min(cur, axis=1)
        am = jnp.min(jnp.where(cur == gm[:, None, :], ci3, BIGI), axis=1)
        gs.append(gm)
        ga.append(am)
        if l < T - 1:
            cur = jnp.where(ci3 == am[:, None, :], INF, cur)
    cols = []
    for _ in range(K):
        m = jnp.min(gs[0], axis=1, keepdims=True)
        j = jnp.min(jnp.where(gs[0] == m, ga[0], BIGI), axis=1)
        cols.append(j)
        mg = ga[0] == j[:, None]
        for l in range(T - 1):
            gs[l] = jnp.where(mg, gs[l + 1], gs[l])
            ga[l] = jnp.where(mg, ga[l + 1], ga[l])
        gs[T - 1] = jnp.where(mg, INF, gs[T - 1])
        ga[T - 1] = jnp.where(mg, BIGI, ga[T - 1])
    col_ref[...] = jnp.stack(cols, axis=1)


def _knn(pos3, qs_pad):
    return pl.pallas_call(
        _knn_body,
        grid=(MP // 128,),
        in_specs=[
            pl.BlockSpec((3, N), lambda i: (0, 0)),
            pl.BlockSpec((128, 3), lambda i: (i, 0)),
        ],
        out_specs=pl.BlockSpec((128, K), lambda i: (i, 0)),
        out_shape=jax.ShapeDtypeStruct((MP, K), jnp.int32),
    )(pos3, qs_pad)


_SC_CH = 128


def _sc_gather(table, idx):
    info = plsc.get_sparse_core_info()
    nw = info.num_cores * info.num_subcores
    b_per_w = EP // nw
    mesh = plsc.VectorSubcoreMesh(core_axis_name="c", subcore_axis_name="s")

    @functools.partial(
        pl.kernel,
        mesh=mesh,
        out_type=jax.ShapeDtypeStruct((EP, DT), jnp.float32),
        scratch_types=[
            pltpu.VMEM((_SC_CH,), jnp.int32),
            pltpu.VMEM((_SC_CH, DT), jnp.float32),
            pltpu.SemaphoreType.DMA,
        ],
    )
    def gk(table_hbm, idx_hbm, out_hbm, idx_v, rows_v, sem):
        wid = lax.axis_index("s") * info.num_cores + lax.axis_index("c")
        base = wid * b_per_w

        def chunk(i, carry):
            off = base + i * _SC_CH
            pltpu.sync_copy(idx_hbm.at[pl.ds(off, _SC_CH)], idx_v)
            pltpu.async_copy(table_hbm.at[idx_v], rows_v, sem).wait()
            pltpu.sync_copy(rows_v, out_hbm.at[pl.ds(off, _SC_CH)])
            return carry

        lax.fori_loop(0, b_per_w // _SC_CH, chunk, 0)

    return gk(table, idx)


_CT = 64


def _conv_body(g_ref, qs_ref, w1_ref, b1_ref, w2_ref, b2_ref, wfr_ref, bfr_ref,
               wq_ref, bq_ref, wk_ref, bk_ref, wv_ref, bv_ref,
               feat_ref, q_ref, k_ref, v_ref):
    g = g_ref[...]
    corr = jnp.dot(qs_ref[...], w1_ref[D_IN:D_IN + 3, :],
                   preferred_element_type=jnp.float32)
    h = jnp.dot(g, w1_ref[...], preferred_element_type=jnp.float32)
    h = h + b1_ref[...]
    h = h.reshape(_CT, K, DIM) - corr[:, None, :]
    h = jnp.maximum(h, 0.0).reshape(_CT * K, DIM)
    h = jnp.dot(h, w2_ref[...], preferred_element_type=jnp.float32) + b2_ref[...]
    agg = jnp.max(h.reshape(_CT, K, 256), axis=1)
    feat = jnp.dot(agg, wfr_ref[...], preferred_element_type=jnp.float32) \
        + bfr_ref[...]
    feat_ref[...] = feat
    q_ref[...] = jnp.dot(feat, wq_ref[...], preferred_element_type=jnp.float32) \
        + bq_ref[...]
    k_ref[...] = jnp.dot(feat, wk_ref[...], preferred_element_type=jnp.float32) \
        + bk_ref[...]
    v_ref[...] = jnp.dot(feat, wv_ref[...], preferred_element_type=jnp.float32) \
        + bv_ref[...]


def _conv(g, qs_pad, w1e, b1, w2, b2, wfr, bfr, wq, bq, wk, bk, wv, bv):
    nt = MP // _CT
    row2 = lambda i: (i, 0)
    full = lambda i: (0, 0)
    shapes = [jax.ShapeDtypeStruct((MP, DIM), jnp.float32)] * 4
    return pl.pallas_call(
        _conv_body,
        grid=(nt,),
        in_specs=[
            pl.BlockSpec((_CT * K, DT), row2),
            pl.BlockSpec((_CT, 3), row2),
            pl.BlockSpec((DT, DIM), full),
            pl.BlockSpec((1, DIM), full),
            pl.BlockSpec((DIM, 256), full),
            pl.BlockSpec((1, 256), full),
            pl.BlockSpec((256, DIM), full),
            pl.BlockSpec((1, DIM), full),
            pl.BlockSpec((DIM, DIM), full),
            pl.BlockSpec((1, DIM), full),
            pl.BlockSpec((DIM, DIM), full),
            pl.BlockSpec((1, DIM), full),
            pl.BlockSpec((DIM, DIM), full),
            pl.BlockSpec((1, DIM), full),
        ],
        out_specs=[pl.BlockSpec((_CT, DIM), row2)] * 4,
        out_shape=shapes,
    )(g, qs_pad, w1e, b1, w2, b2, wfr, bfr, wq, bq, wk, bk, wv, bv)


def _ln(x, g, b):
    mu = jnp.mean(x, axis=-1, keepdims=True)
    v = jnp.mean((x - mu) ** 2, axis=-1, keepdims=True)
    return (x - mu) / jnp.sqrt(v + 1e-5) * g + b


def _attn_body(q_ref, k_ref, v_ref, feat_ref, wo_ref, bo_ref, g1_ref, be1_ref,
               wm1_ref, bm1_ref, wm2_ref, bm2_ref, g2_ref, be2_ref, out_ref):
    qt = q_ref[...]
    kf = k_ref[...]
    vf = v_ref[...]
    valid = lax.broadcasted_iota(jnp.int32, (128, MP), 1) < M
    scale = 1.0 / jnp.sqrt(jnp.float32(DH))
    heads = []
    for h in range(HEADS):
        qh = qt[:, h * DH:(h + 1) * DH]
        kh = kf[:, h * DH:(h + 1) * DH]
        vh = vf[:, h * DH:(h + 1) * DH]
        s = lax.dot_general(qh, kh, (((1,), (1,)), ((), ())),
                            preferred_element_type=jnp.float32) * scale
        s = jnp.where(valid, s, NEG)
        s = s - jnp.max(s, axis=1, keepdims=True)
        e = jnp.exp(s)
        a = e / jnp.sum(e, axis=1, keepdims=True)
        heads.append(jnp.dot(a, vh, preferred_element_type=jnp.float32))
    att = jnp.concatenate(heads, axis=1)
    att = jnp.dot(att, wo_ref[...], preferred_element_type=jnp.float32) \
        + bo_ref[...]
    x1 = _ln(feat_ref[...] + att, g1_ref[...], be1_ref[...])
    mlp = jnp.maximum(
        jnp.dot(x1, wm1_ref[...], preferred_element_type=jnp.float32)
        + bm1_ref[...], 0.0)
    mlp = jnp.dot(mlp, wm2_ref[...], preferred_element_type=jnp.float32) \
        + bm2_ref[...]
    out_ref[...] = _ln(x1 + mlp, g2_ref[...], be2_ref[...])


def _attn(q, k, v, feat, wo, bo, g1, be1, wm1, bm1, wm2, bm2, g2, be2):
    row = lambda i: (i, 0)
    full = lambda i: (0, 0)
    return pl.pallas_call(
        _attn_body,
        grid=(MP // 128,),
        in_specs=[
            pl.BlockSpec((128, DIM), row),
            pl.BlockSpec((MP, DIM), full),
            pl.BlockSpec((MP, DIM), full),
            pl.BlockSpec((128, DIM), row),
            pl.BlockSpec((DIM, DIM), full),
            pl.BlockSpec((1, DIM), full),
            pl.BlockSpec((1, DIM), full),
            pl.BlockSpec((1, DIM), full),
            pl.BlockSpec((DIM, 2 * DIM), full),
            pl.BlockSpec((1, 2 * DIM), full),
            pl.BlockSpec((2 * DIM, DIM), full),
            pl.BlockSpec((1, DIM), full),
            pl.BlockSpec((1, DIM), full),
            pl.BlockSpec((1, DIM), full),
        ],
        out_specs=pl.BlockSpec((128, DIM), row),
        out_shape=jax.ShapeDtypeStruct((MP, DIM), jnp.float32),
    )(q, k, v, feat, wo, bo, g1, be1, wm1, bm1, wm2, bm2, g2, be2)


def kernel(x, pos, batch, W1, b1, W2, b2, Wfr, bfr, Wq, bq, Wk, bk, Wv, bv,
           Wo, bo, g1, be1, Wm1, bm1, Wm2, bm2, g2, be2):
    combined = _fps(pos)
    pos_s = pos[combined]
    qs_pad = jnp.zeros((MP, 3), jnp.float32).at[:M].set(pos_s)

    pos3 = pos.T
    col = _knn(pos3, qs_pad)

    table = jnp.concatenate(
        [x, pos, jnp.zeros((N, DT - D_IN - 3), jnp.float32)], axis=1)
    g = _sc_gather(table, col.reshape(-1))

    w1e = jnp.concatenate([W1, jnp.zeros((DT - D_IN - 3, DIM), jnp.float32)],
                          axis=0)
    r2 = lambda a: a.reshape(1, -1)
    feat, q, k, v = _conv(g, qs_pad, w1e, r2(b1), W2, r2(b2), Wfr, r2(bfr),
                          Wq, r2(bq), Wk, r2(bk), Wv, r2(bv))
    x2 = _attn(q, k, v, feat, Wo, r2(bo), r2(g1), r2(be1), Wm1, r2(bm1),
               Wm2, r2(bm2), r2(g2), r2(be2))
    return (x2[:M], pos_s, batch[combined])


# speedup vs baseline: 1.4533x; 1.4533x over previous
import functools
import jax
import jax.numpy as jnp
from jax import lax
from jax.experimental import pallas as pl
from jax.experimental.pallas import tpu as pltpu
from jax.experimental.pallas import tpu_sc as plsc

N = 16384
D_IN = 64
K = 32
DIM = 128
HEADS = 4
DH = DIM // HEADS
M = 2457
N_BASE = 2048
N_EXTRA = 409
MP = 2560
EP = MP * K
DT = 128
NEG = -1e30
INF = 3e38
BIGI = 2**30


def _fps_body(px_ref, py_ref, pz_ref, idx_ref):
    px = px_ref[...]
    py = py_ref[...]
    pz = pz_ref[...]
    row = lax.broadcasted_iota(jnp.int32, (128, 128), 0)
    col = lax.broadcasted_iota(jnp.int32, (128, 128), 1)
    iota = row * 128 + col

    def init_d(start):
        sx = px[start // 128, start % 128]
        sy = py[start // 128, start % 128]
        sz = pz[start // 128, start % 128]
        return ((px - sx) ** 2 + (pz - sz) ** 2) + (py - sy) ** 2

    def step(d, i, out_off):
        m = jnp.max(d)
        nxt = jnp.min(jnp.where(d == m, iota, BIGI))
        sel = iota == nxt
        nx = jnp.sum(jnp.where(sel, px, 0.0))
        ny = jnp.sum(jnp.where(sel, py, 0.0))
        nz = jnp.sum(jnp.where(sel, pz, 0.0))
        idx_ref[out_off + i] = nxt
        dn = ((px - nx) ** 2 + (pz - nz) ** 2) + (py - ny) ** 2
        return jnp.minimum(d, dn)

    idx_ref[0] = jnp.int32(0)
    idx_ref[N_BASE] = jnp.int32(N // 2)
    d1 = init_d(0)
    d2 = init_d(N // 2)

    def both(i, carry):
        a, b = carry
        return (step(a, i, 0), step(b, i, N_BASE))

    d1, d2 = lax.fori_loop(1, N_EXTRA, both, (d1, d2))
    lax.fori_loop(N_EXTRA, N_BASE, lambda i, d: step(d, i, 0), d1)


def _fps(pos):
    px = pos[:, 0].reshape(128, 128)
    py = pos[:, 1].reshape(128, 128)
    pz = pos[:, 2].reshape(128, 128)
    idx = pl.pallas_call(
        _fps_body,
        out_shape=jax.ShapeDtypeStruct((2464,), jnp.int32),
        out_specs=pl.BlockSpec(memory_space=pltpu.SMEM),
    )(px, py, pz)
    return idx[:M]


def _knn_body(pos3_ref, qs_ref, col_ref):
    px = pos3_ref[0, :]
    py = pos3_ref[1, :]
    pz = pos3_ref[2, :]
    qx = qs_ref[:, 0]
    qy = qs_ref[:, 1]
    qz = qs_ref[:, 2]
    qn = (qx * qx + qz * qz) + qy * qy
    pn = (px * px + pz * pz) + py * py
    bf = lambda a: a.astype(jnp.bfloat16).astype(jnp.float32)
    qbx, qby, qbz = bf(qx), bf(qy), bf(qz)
    pbx, pby, pbz = bf(px), bf(py), bf(pz)
    t = (qbx[:, None] * pbx[None, :] + qby[:, None] * pby[None, :]) \
        + qbz[:, None] * pbz[None, :]
    d = (qn[:, None] + pn[None, :]) - 2.0 * t
    ci = lax.broadcasted_iota(jnp.int32, (128, N), 1)
    cols = []
    for _ in range(K):
        m = jnp.min(d, axis=1, keepdims=True)
        idx_k = jnp.min(jnp.where(d == m, ci, BIGI), axis=1)
        cols.append(idx_k)
        d = jnp.where(ci == idx_k[:, None], INF, d)
    col_ref[...] = jnp.stack(cols, axis=1)


def _knn(pos3, qs_pad):
    return pl.pallas_call(
        _knn_body,
        grid=(MP // 128,),
        in_specs=[
            pl.BlockSpec((3, N), lambda i: (0, 0)),
            pl.BlockSpec((128, 3), lambda i: (i, 0)),
        ],
        out_specs=pl.BlockSpec((128, K), lambda i: (i, 0)),
        out_shape=jax.ShapeDtypeStruct((MP, K), jnp.int32),
    )(pos3, qs_pad)


_SC_CH = 128


def _sc_gather(table, idx):
    info = plsc.get_sparse_core_info()
    nw = info.num_cores * info.num_subcores
    b_per_w = EP // nw
    mesh = plsc.VectorSubcoreMesh(core_axis_name="c", subcore_axis_name="s")

    @functools.partial(
        pl.kernel,
        mesh=mesh,
        out_type=jax.ShapeDtypeStruct((EP, DT), jnp.float32),
        scratch_types=[
            pltpu.VMEM((_SC_CH,), jnp.int32),
            pltpu.VMEM((_SC_CH, DT), jnp.float32),
            pltpu.SemaphoreType.DMA,
        ],
    )
    def gk(table_hbm, idx_hbm, out_hbm, idx_v, rows_v, sem):
        wid = lax.axis_index("s") * info.num_cores + lax.axis_index("c")
        base = wid * b_per_w

        def chunk(i, carry):
            off = base + i * _SC_CH
            pltpu.sync_copy(idx_hbm.at[pl.ds(off, _SC_CH)], idx_v)
            pltpu.async_copy(table_hbm.at[idx_v], rows_v, sem).wait()
            pltpu.sync_copy(rows_v, out_hbm.at[pl.ds(off, _SC_CH)])
            return carry

        lax.fori_loop(0, b_per_w // _SC_CH, chunk, 0)

    return gk(table, idx)


_CT = 64


def _conv_body(g_ref, qs_ref, w1_ref, b1_ref, w2_ref, b2_ref, wfr_ref, bfr_ref,
               wq_ref, bq_ref, wk_ref, bk_ref, wv_ref, bv_ref,
               feat_ref, q_ref, k_ref, v_ref):
    g = g_ref[...]
    corr = jnp.dot(qs_ref[...], w1_ref[D_IN:D_IN + 3, :],
                   preferred_element_type=jnp.float32)
    h = jnp.dot(g, w1_ref[...], preferred_element_type=jnp.float32)
    h = h + b1_ref[...]
    h = h.reshape(_CT, K, DIM) - corr[:, None, :]
    h = jnp.maximum(h, 0.0).reshape(_CT * K, DIM)
    h = jnp.dot(h, w2_ref[...], preferred_element_type=jnp.float32) + b2_ref[...]
    agg = jnp.max(h.reshape(_CT, K, 256), axis=1)
    feat = jnp.dot(agg, wfr_ref[...], preferred_element_type=jnp.float32) \
        + bfr_ref[...]
    feat_ref[...] = feat
    q_ref[...] = jnp.dot(feat, wq_ref[...], preferred_element_type=jnp.float32) \
        + bq_ref[...]
    k_ref[...] = jnp.dot(feat, wk_ref[...], preferred_element_type=jnp.float32) \
        + bk_ref[...]
    v_ref[...] = jnp.dot(feat, wv_ref[...], preferred_element_type=jnp.float32) \
        + bv_ref[...]


def _conv(g, qs_pad, w1e, b1, w2, b2, wfr, bfr, wq, bq, wk, bk, wv, bv):
    nt = MP // _CT
    row2 = lambda i: (i, 0)
    full = lambda i: (0, 0)
    shapes = [jax.ShapeDtypeStruct((MP, DIM), jnp.float32)] * 4
    return pl.pallas_call(
        _conv_body,
        grid=(nt,),
        in_specs=[
            pl.BlockSpec((_CT * K, DT), row2),
            pl.BlockSpec((_CT, 3), row2),
            pl.BlockSpec((DT, DIM), full),
            pl.BlockSpec((1, DIM), full),
            pl.BlockSpec((DIM, 256), full),
            pl.BlockSpec((1, 256), full),
            pl.BlockSpec((256, DIM), full),
            pl.BlockSpec((1, DIM), full),
            pl.BlockSpec((DIM, DIM), full),
            pl.BlockSpec((1, DIM), full),
            pl.BlockSpec((DIM, DIM), full),
            pl.BlockSpec((1, DIM), full),
            pl.BlockSpec((DIM, DIM), full),
            pl.BlockSpec((1, DIM), full),
        ],
        out_specs=[pl.BlockSpec((_CT, DIM), row2)] * 4,
        out_shape=shapes,
    )(g, qs_pad, w1e, b1, w2, b2, wfr, bfr, wq, bq, wk, bk, wv, bv)


def _ln(x, g, b):
    mu = jnp.mean(x, axis=-1, keepdims=True)
    v = jnp.mean((x - mu) ** 2, axis=-1, keepdims=True)
    return (x - mu) / jnp.sqrt(v + 1e-5) * g + b


def _attn_body(q_ref, k_ref, v_ref, feat_ref, wo_ref, bo_ref, g1_ref, be1_ref,
               wm1_ref, bm1_ref, wm2_ref, bm2_ref, g2_ref, be2_ref, out_ref):
    qt = q_ref[...]
    kf = k_ref[...]
    vf = v_ref[...]
    valid = lax.broadcasted_iota(jnp.int32, (128, MP), 1) < M
    scale = 1.0 / jnp.sqrt(jnp.float32(DH))
    heads = []
    for h in range(HEADS):
        qh = qt[:, h * DH:(h + 1) * DH]
        kh = kf[:, h * DH:(h + 1) * DH]
        vh = vf[:, h * DH:(h + 1) * DH]
        s = lax.dot_general(qh, kh, (((1,), (1,)), ((), ())),
                            preferred_element_type=jnp.float32) * scale
        s = jnp.where(valid, s, NEG)
        s = s - jnp.max(s, axis=1, keepdims=True)
        e = jnp.exp(s)
        a = e / jnp.sum(e, axis=1, keepdims=True)
        heads.append(jnp.dot(a, vh, preferred_element_type=jnp.float32))
    att = jnp.concatenate(heads, axis=1)
    att = jnp.dot(att, wo_ref[...], preferred_element_type=jnp.float32) \
        + bo_ref[...]
    x1 = _ln(feat_ref[...] + att, g1_ref[...], be1_ref[...])
    mlp = jnp.maximum(
        jnp.dot(x1, wm1_ref[...], preferred_element_type=jnp.float32)
        + bm1_ref[...], 0.0)
    mlp = jnp.dot(mlp, wm2_ref[...], preferred_element_type=jnp.float32) \
        + bm2_ref[...]
    out_ref[...] = _ln(x1 + mlp, g2_ref[...], be2_ref[...])


def _attn(q, k, v, feat, wo, bo, g1, be1, wm1, bm1, wm2, bm2, g2, be2):
    row = lambda i: (i, 0)
    full = lambda i: (0, 0)
    return pl.pallas_call(
        _attn_body,
        grid=(MP // 128,),
        in_specs=[
            pl.BlockSpec((128, DIM), row),
            pl.BlockSpec((MP, DIM), full),
            pl.BlockSpec((MP, DIM), full),
            pl.BlockSpec((128, DIM), row),
            pl.BlockSpec((DIM, DIM), full),
            pl.BlockSpec((1, DIM), full),
            pl.BlockSpec((1, DIM), full),
            pl.BlockSpec((1, DIM), full),
            pl.BlockSpec((DIM, 2 * DIM), full),
            pl.BlockSpec((1, 2 * DIM), full),
            pl.BlockSpec((2 * DIM, DIM), full),
            pl.BlockSpec((1, DIM), full),
            pl.BlockSpec((1, DIM), full),
            pl.BlockSpec((1, DIM), full),
        ],
        out_specs=pl.BlockSpec((128, DIM), row),
        out_shape=jax.ShapeDtypeStruct((MP, DIM), jnp.float32),
    )(q, k, v, feat, wo, bo, g1, be1, wm1, bm1, wm2, bm2, g2, be2)


def kernel(x, pos, batch, W1, b1, W2, b2, Wfr, bfr, Wq, bq, Wk, bk, Wv, bv,
           Wo, bo, g1, be1, Wm1, bm1, Wm2, bm2, g2, be2):
    combined = _fps(pos)
    pos_s = pos[combined]
    qs_pad = jnp.zeros((MP, 3), jnp.float32).at[:M].set(pos_s)

    pos3 = pos.T
    col = _knn(pos3, qs_pad)

    table = jnp.concatenate(
        [x, pos, jnp.zeros((N, DT - D_IN - 3), jnp.float32)], axis=1)
    g = _sc_gather(table, col.reshape(-1))

    w1e = jnp.concatenate([W1, jnp.zeros((DT - D_IN - 3, DIM), jnp.float32)],
                          axis=0)
    r2 = lambda a: a.reshape(1, -1)
    feat, q, k, v = _conv(g, qs_pad, w1e, r2(b1), W2, r2(b2), Wfr, r2(bfr),
                          Wq, r2(bq), Wk, r2(bk), Wv, r2(bv))
    x2 = _attn(q, k, v, feat, Wo, r2(bo), r2(g1), r2(be1), Wm1, r2(bm1),
               Wm2, r2(bm2), r2(g2), r2(be2))
    return (x2[:M], pos_s, batch[combined])


# speedup vs baseline: 1.4915x; 1.0263x over previous
import functools
import jax
import jax.numpy as jnp
from jax import lax
from jax.experimental import pallas as pl
from jax.experimental.pallas import tpu as pltpu
from jax.experimental.pallas import tpu_sc as plsc

N = 16384
D_IN = 64
K = 32
DIM = 128
HEADS = 4
DH = DIM // HEADS
M = 2457
N_BASE = 2048
N_EXTRA = 409
MP = 2560
EP = MP * K
DT = 128
NEG = -1e30
INF = 3e38
BIGI = 2**30


def _fps_body(px_ref, py_ref, pz_ref, idx_ref):
    px = px_ref[...]
    py = py_ref[...]
    pz = pz_ref[...]
    row = lax.broadcasted_iota(jnp.int32, (128, 128), 0)
    col = lax.broadcasted_iota(jnp.int32, (128, 128), 1)
    iota = row * 128 + col

    def init_d(start):
        sx = px[start // 128, start % 128]
        sy = py[start // 128, start % 128]
        sz = pz[start // 128, start % 128]
        return ((px - sx) ** 2 + (pz - sz) ** 2) + (py - sy) ** 2

    def step(d, i, out_off):
        m = jnp.max(d)
        nxt = jnp.min(jnp.where(d == m, iota, BIGI))
        sel = iota == nxt
        nx = jnp.sum(jnp.where(sel, px, 0.0))
        ny = jnp.sum(jnp.where(sel, py, 0.0))
        nz = jnp.sum(jnp.where(sel, pz, 0.0))
        idx_ref[out_off + i] = nxt
        dn = ((px - nx) ** 2 + (pz - nz) ** 2) + (py - ny) ** 2
        return jnp.minimum(d, dn)

    idx_ref[0] = jnp.int32(0)
    idx_ref[N_BASE] = jnp.int32(N // 2)
    d1 = init_d(0)
    d2 = init_d(N // 2)

    def both(i, carry):
        a, b = carry
        return (step(a, i, 0), step(b, i, N_BASE))

    d1, d2 = lax.fori_loop(1, N_EXTRA, both, (d1, d2))
    lax.fori_loop(N_EXTRA, N_BASE, lambda i, d: step(d, i, 0), d1)


def _fps(pos):
    px = pos[:, 0].reshape(128, 128)
    py = pos[:, 1].reshape(128, 128)
    pz = pos[:, 2].reshape(128, 128)
    idx = pl.pallas_call(
        _fps_body,
        out_shape=jax.ShapeDtypeStruct((2464,), jnp.int32),
        out_specs=pl.BlockSpec(memory_space=pltpu.SMEM),
    )(px, py, pz)
    return idx[:M]


def _knn_body(pos3_ref, qs_ref, col_ref):
    px = pos3_ref[0, :]
    py = pos3_ref[1, :]
    pz = pos3_ref[2, :]
    qx = qs_ref[:, 0]
    qy = qs_ref[:, 1]
    qz = qs_ref[:, 2]
    qn = (qx * qx + qz * qz) + qy * qy
    pn = (px * px + pz * pz) + py * py
    bf = lambda a: a.astype(jnp.bfloat16).astype(jnp.float32)
    qbx, qby, qbz = bf(qx), bf(qy), bf(qz)
    pbx, pby, pbz = bf(px), bf(py), bf(pz)
    t = (qbx[:, None] * pbx[None, :] + qby[:, None] * pby[None, :]) \
        + qbz[:, None] * pbz[None, :]
    d = (qn[:, None] + pn[None, :]) - 2.0 * t
    ci = lax.broadcasted_iota(jnp.int32, (128, N), 1)
    cols = []
    for _ in range(K):
        idx_k = jnp.argmin(d, axis=1).astype(jnp.int32)
        cols.append(idx_k)
        d = jnp.where(ci == idx_k[:, None], INF, d)
    col_ref[...] = jnp.stack(cols, axis=1)


def _knn(pos3, qs_pad):
    return pl.pallas_call(
        _knn_body,
        grid=(MP // 128,),
        in_specs=[
            pl.BlockSpec((3, N), lambda i: (0, 0)),
            pl.BlockSpec((128, 3), lambda i: (i, 0)),
        ],
        out_specs=pl.BlockSpec((128, K), lambda i: (i, 0)),
        out_shape=jax.ShapeDtypeStruct((MP, K), jnp.int32),
    )(pos3, qs_pad)


_SC_CH = 128


def _sc_gather(table, idx):
    info = plsc.get_sparse_core_info()
    nw = info.num_cores * info.num_subcores
    b_per_w = EP // nw
    mesh = plsc.VectorSubcoreMesh(core_axis_name="c", subcore_axis_name="s")

    @functools.partial(
        pl.kernel,
        mesh=mesh,
        out_type=jax.ShapeDtypeStruct((EP, DT), jnp.float32),
        scratch_types=[
            pltpu.VMEM((_SC_CH,), jnp.int32),
            pltpu.VMEM((_SC_CH, DT), jnp.float32),
            pltpu.SemaphoreType.DMA,
        ],
    )
    def gk(table_hbm, idx_hbm, out_hbm, idx_v, rows_v, sem):
        wid = lax.axis_index("s") * info.num_cores + lax.axis_index("c")
        base = wid * b_per_w

        def chunk(i, carry):
            off = base + i * _SC_CH
            pltpu.sync_copy(idx_hbm.at[pl.ds(off, _SC_CH)], idx_v)
            pltpu.async_copy(table_hbm.at[idx_v], rows_v, sem).wait()
            pltpu.sync_copy(rows_v, out_hbm.at[pl.ds(off, _SC_CH)])
            return carry

        lax.fori_loop(0, b_per_w // _SC_CH, chunk, 0)

    return gk(table, idx)


_CT = 64


def _conv_body(g_ref, qs_ref, w1_ref, b1_ref, w2_ref, b2_ref, wfr_ref, bfr_ref,
               wq_ref, bq_ref, wk_ref, bk_ref, wv_ref, bv_ref,
               feat_ref, q_ref, k_ref, v_ref):
    g = g_ref[...]
    corr = jnp.dot(qs_ref[...], w1_ref[D_IN:D_IN + 3, :],
                   preferred_element_type=jnp.float32)
    h = jnp.dot(g, w1_ref[...], preferred_element_type=jnp.float32)
    h = h + b1_ref[...]
    h = h.reshape(_CT, K, DIM) - corr[:, None, :]
    h = jnp.maximum(h, 0.0).reshape(_CT * K, DIM)
    h = jnp.dot(h, w2_ref[...], preferred_element_type=jnp.float32) + b2_ref[...]
    agg = jnp.max(h.reshape(_CT, K, 256), axis=1)
    feat = jnp.dot(agg, wfr_ref[...], preferred_element_type=jnp.float32) \
        + bfr_ref[...]
    feat_ref[...] = feat
    q_ref[...] = jnp.dot(feat, wq_ref[...], preferred_element_type=jnp.float32) \
        + bq_ref[...]
    k_ref[...] = jnp.dot(feat, wk_ref[...], preferred_element_type=jnp.float32) \
        + bk_ref[...]
    v_ref[...] = jnp.dot(feat, wv_ref[...], preferred_element_type=jnp.float32) \
        + bv_ref[...]


def _conv(g, qs_pad, w1e, b1, w2, b2, wfr, bfr, wq, bq, wk, bk, wv, bv):
    nt = MP // _CT
    row2 = lambda i: (i, 0)
    full = lambda i: (0, 0)
    shapes = [jax.ShapeDtypeStruct((MP, DIM), jnp.float32)] * 4
    return pl.pallas_call(
        _conv_body,
        grid=(nt,),
        in_specs=[
            pl.BlockSpec((_CT * K, DT), row2),
            pl.BlockSpec((_CT, 3), row2),
            pl.BlockSpec((DT, DIM), full),
            pl.BlockSpec((1, DIM), full),
            pl.BlockSpec((DIM, 256), full),
            pl.BlockSpec((1, 256), full),
            pl.BlockSpec((256, DIM), full),
            pl.BlockSpec((1, DIM), full),
            pl.BlockSpec((DIM, DIM), full),
            pl.BlockSpec((1, DIM), full),
            pl.BlockSpec((DIM, DIM), full),
            pl.BlockSpec((1, DIM), full),
            pl.BlockSpec((DIM, DIM), full),
            pl.BlockSpec((1, DIM), full),
        ],
        out_specs=[pl.BlockSpec((_CT, DIM), row2)] * 4,
        out_shape=shapes,
    )(g, qs_pad, w1e, b1, w2, b2, wfr, bfr, wq, bq, wk, bk, wv, bv)


def _ln(x, g, b):
    mu = jnp.mean(x, axis=-1, keepdims=True)
    v = jnp.mean((x - mu) ** 2, axis=-1, keepdims=True)
    return (x - mu) / jnp.sqrt(v + 1e-5) * g + b


def _attn_body(q_ref, k_ref, v_ref, feat_ref, wo_ref, bo_ref, g1_ref, be1_ref,
               wm1_ref, bm1_ref, wm2_ref, bm2_ref, g2_ref, be2_ref, out_ref):
    qt = q_ref[...]
    kf = k_ref[...]
    vf = v_ref[...]
    valid = lax.broadcasted_iota(jnp.int32, (128, MP), 1) < M
    scale = 1.0 / jnp.sqrt(jnp.float32(DH))
    heads = []
    for h in range(HEADS):
        qh = qt[:, h * DH:(h + 1) * DH]
        kh = kf[:, h * DH:(h + 1) * DH]
        vh = vf[:, h * DH:(h + 1) * DH]
        s = lax.dot_general(qh, kh, (((1,), (1,)), ((), ())),
                            preferred_element_type=jnp.float32) * scale
        s = jnp.where(valid, s, NEG)
        s = s - jnp.max(s, axis=1, keepdims=True)
        e = jnp.exp(s)
        a = e / jnp.sum(e, axis=1, keepdims=True)
        heads.append(jnp.dot(a, vh, preferred_element_type=jnp.float32))
    att = jnp.concatenate(heads, axis=1)
    att = jnp.dot(att, wo_ref[...], preferred_element_type=jnp.float32) \
        + bo_ref[...]
    x1 = _ln(feat_ref[...] + att, g1_ref[...], be1_ref[...])
    mlp = jnp.maximum(
        jnp.dot(x1, wm1_ref[...], preferred_element_type=jnp.float32)
        + bm1_ref[...], 0.0)
    mlp = jnp.dot(mlp, wm2_ref[...], preferred_element_type=jnp.float32) \
        + bm2_ref[...]
    out_ref[...] = _ln(x1 + mlp, g2_ref[...], be2_ref[...])


def _attn(q, k, v, feat, wo, bo, g1, be1, wm1, bm1, wm2, bm2, g2, be2):
    row = lambda i: (i, 0)
    full = lambda i: (0, 0)
    return pl.pallas_call(
        _attn_body,
        grid=(MP // 128,),
        in_specs=[
            pl.BlockSpec((128, DIM), row),
            pl.BlockSpec((MP, DIM), full),
            pl.BlockSpec((MP, DIM), full),
            pl.BlockSpec((128, DIM), row),
            pl.BlockSpec((DIM, DIM), full),
            pl.BlockSpec((1, DIM), full),
            pl.BlockSpec((1, DIM), full),
            pl.BlockSpec((1, DIM), full),
            pl.BlockSpec((DIM, 2 * DIM), full),
            pl.BlockSpec((1, 2 * DIM), full),
            pl.BlockSpec((2 * DIM, DIM), full),
            pl.BlockSpec((1, DIM), full),
            pl.BlockSpec((1, DIM), full),
            pl.BlockSpec((1, DIM), full),
        ],
        out_specs=pl.BlockSpec((128, DIM), row),
        out_shape=jax.ShapeDtypeStruct((MP, DIM), jnp.float32),
    )(q, k, v, feat, wo, bo, g1, be1, wm1, bm1, wm2, bm2, g2, be2)


def kernel(x, pos, batch, W1, b1, W2, b2, Wfr, bfr, Wq, bq, Wk, bk, Wv, bv,
           Wo, bo, g1, be1, Wm1, bm1, Wm2, bm2, g2, be2):
    combined = _fps(pos)
    pos_s = pos[combined]
    qs_pad = jnp.zeros((MP, 3), jnp.float32).at[:M].set(pos_s)

    pos3 = pos.T
    col = _knn(pos3, qs_pad)

    table = jnp.concatenate(
        [x, pos, jnp.zeros((N, DT - D_IN - 3), jnp.float32)], axis=1)
    g = _sc_gather(table, col.reshape(-1))

    w1e = jnp.concatenate([W1, jnp.zeros((DT - D_IN - 3, DIM), jnp.float32)],
                          axis=0)
    r2 = lambda a: a.reshape(1, -1)
    feat, q, k, v = _conv(g, qs_pad, w1e, r2(b1), W2, r2(b2), Wfr, r2(bfr),
                          Wq, r2(bq), Wk, r2(bk), Wv, r2(bv))
    x2 = _attn(q, k, v, feat, Wo, r2(bo), r2(g1), r2(be1), Wm1, r2(bm1),
               Wm2, r2(bm2), r2(g2), r2(be2))
    return (x2[:M], pos_s, batch[combined])
